# baseline (speedup 1.0000x reference)
import jax
import jax.numpy as jnp
from jax.experimental import pallas as pl
from jax.experimental.pallas import tpu as pltpu

DEPTH = 12
N = 2 ** DEPTH - 1
B = 8
H = 128
D = 128
CH_NODES = 256

_CHUNKS = []
for _d in range(DEPTH - 1, -1, -1):
    _n = 2 ** _d
    for _p0 in range(0, _n, CH_NODES):
        _CHUNKS.append((_d, _p0, min(CH_NODES, _n - _p0)))


def _dot(a, b):
    return jax.lax.dot_general(
        a, b, (((1,), (0,)), ((), ())),
        preferred_element_type=jnp.float32)


def _sig(x):
    return 0.5 * jnp.tanh(0.5 * x) + 0.5


def _pairsum(v, cn):
    v3 = v.reshape(cn, 2 * B, H)
    return (v3[:, :B, :] + v3[:, B:, :]).reshape(cn * B, H)


def _x_copy(feat_hbm, xbuf, x_sem, k):
    d, p0, cn = _CHUNKS[k]
    g0 = (2 ** d - 1) + p0
    slot = k % 2
    return [
        pltpu.make_async_copy(
            feat_hbm.at[b, pl.ds(g0, cn), :],
            xbuf.at[slot, pl.ds(0, cn), b, :],
            x_sem.at[slot, b])
        for b in range(B)
    ]


def _tree_kernel(feat_hbm, wiou_ref, biou_ref, uiou_ref, wf_ref, bf_ref,
                 uf_ref, h_hbm, c_hbm, xbuf, hbuf, cbuf, x_sem, out_sem):
    for cp in _x_copy(feat_hbm, xbuf, x_sem, 0):
        cp.start()
    n_out = 0
    for k, (d, p0, cn) in enumerate(_CHUNKS):
        lvl0 = 2 ** d - 1
        g0 = lvl0 + p0
        slot = k % 2
        for cp in _x_copy(feat_hbm, xbuf, x_sem, k):
            cp.wait()
        if k + 1 < len(_CHUNKS):
            for cp in _x_copy(feat_hbm, xbuf, x_sem, k + 1):
                cp.start()
        cr = cn * B
        x = xbuf[slot, pl.ds(0, cn), :, :].reshape(cr, D)
        x = x.astype(jnp.bfloat16)
        iou = _dot(x, wiou_ref[:, :]) + biou_ref[0, :]
        if d < DEPTH - 1:
            ch0 = (2 ** (d + 1) - 1) + 2 * p0
            ch = hbuf[pl.ds(ch0, 2 * cn), :, :].reshape(2 * cr, H)
            ch = ch.astype(jnp.bfloat16)
            cc = cbuf[pl.ds(ch0, 2 * cn), :, :].reshape(2 * cr, H)
            hs = _pairsum(ch, cn)
            cs = _pairsum(cc, cn)
            iou = iou + _dot(hs, uiou_ref[:, :])
            fx = _dot(x, wf_ref[:, :]) + bf_ref[0, :]
            g = _dot(ch, uf_ref[:, :]) * cc
            gs = _pairsum(g, cn)
        i = _sig(iou[:, :H])
        o = _sig(iou[:, H:2 * H])
        u = jnp.tanh(iou[:, 2 * H:])
        c = i * u
        if d < DEPTH - 1:
            c = c + fx * cs + gs
        h = o * jnp.tanh(c)
        hbuf[pl.ds(g0, cn), :, :] = h.reshape(cn, B, H)
        cbuf[pl.ds(g0, cn), :, :] = c.reshape(cn, B, H)
        for b in range(B):
            pltpu.make_async_copy(
                hbuf.at[pl.ds(g0, cn), b, :],
                h_hbm.at[b, pl.ds(g0, cn), :], out_sem).start()
            pltpu.make_async_copy(
                cbuf.at[pl.ds(g0, cn), b, :],
                c_hbm.at[b, pl.ds(g0, cn), :], out_sem).start()
            n_out += 2
    for d, p0, cn in _CHUNKS:
        g0 = (2 ** d - 1) + p0
        for b in range(B):
            pltpu.make_async_copy(
                hbuf.at[pl.ds(g0, cn), b, :],
                h_hbm.at[b, pl.ds(g0, cn), :], out_sem).wait()
            pltpu.make_async_copy(
                cbuf.at[pl.ds(g0, cn), b, :],
                c_hbm.at[b, pl.ds(g0, cn), :], out_sem).wait()


def kernel(features, descendants, parents, W_iou, b_iou, U_iou, W_f, b_f,
           U_f):
    del descendants, parents
    hbm = pl.BlockSpec(memory_space=pltpu.MemorySpace.HBM)
    vmem = pl.BlockSpec(memory_space=pltpu.MemorySpace.VMEM)
    h, c = pl.pallas_call(
        _tree_kernel,
        in_specs=[hbm] + [vmem] * 6,
        out_specs=[hbm, hbm],
        out_shape=[jax.ShapeDtypeStruct((B, N, H), jnp.float32)] * 2,
        scratch_shapes=[
            pltpu.MemorySpace.VMEM((2, CH_NODES, B, D), jnp.float32),
            pltpu.MemorySpace.VMEM((N, B, H), jnp.float32),
            pltpu.MemorySpace.VMEM((N, B, H), jnp.float32),
            pltpu.SemaphoreType.DMA((2, B)),
            pltpu.SemaphoreType.DMA,
        ],
    )(features, W_iou.T.astype(jnp.bfloat16), b_iou.reshape(1, -1),
      U_iou.T.astype(jnp.bfloat16), W_f.T.astype(jnp.bfloat16),
      b_f.reshape(1, -1), U_f.T.astype(jnp.bfloat16))
    return (h, c)


# speedup vs baseline: 2.3852x; 2.3852x over previous
import jax
import jax.numpy as jnp
from jax.experimental import pallas as pl

DEPTH = 12
N = 2 ** DEPTH - 1


def _dot(a, b):
    return jax.lax.dot_general(
        a, b, (((1,), (0,)), ((), ())),
        preferred_element_type=jnp.float32)


def _sig(x):
    return 0.5 * jnp.tanh(x) + 0.5


def _tree_kernel(feat_ref, wiou_ref, biou_ref, uiou_ref, wf_ref, bf_ref,
                 uf_ref, h_ref, c_ref):
    B = feat_ref.shape[0] // N
    H = uf_ref.shape[0]
    CH = 2048

    for d in range(DEPTH - 1, -1, -1):
        n = 2 ** d
        rows = n * B
        r0 = (n - 1) * B
        cb = (2 * n - 1) * B
        for c0 in range(0, rows, CH):
            cr = min(CH, rows - c0)
            x = feat_ref[pl.ds(r0 + c0, cr), :].astype(jnp.bfloat16)
            iou = _dot(x, wiou_ref[:, :]) + biou_ref[0, :]
            if d < DEPTH - 1:
                ch = h_ref[pl.ds(cb + 2 * c0, 2 * cr), :].astype(jnp.bfloat16)
                cc = c_ref[pl.ds(cb + 2 * c0, 2 * cr), :]
                ch3 = ch.reshape(cr // B, 2 * B, H)
                hs = (ch3[:, :B, :] + ch3[:, B:, :]).reshape(cr, H)
                cc3 = cc.reshape(cr // B, 2 * B, H)
                cs = (cc3[:, :B, :] + cc3[:, B:, :]).reshape(cr, H)
                iou = iou + _dot(hs, uiou_ref[:, :])
                fx = _dot(x, wf_ref[:, :]) + bf_ref[0, :]
                g = _dot(ch, uf_ref[:, :]) * cc
                g3 = g.reshape(cr // B, 2 * B, H)
                gs = (g3[:, :B, :] + g3[:, B:, :]).reshape(cr, H)
            i = _sig(iou[:, :H])
            o = _sig(iou[:, H:2 * H])
            u = jnp.tanh(iou[:, 2 * H:])
            c = i * u
            if d < DEPTH - 1:
                c = c + fx * cs + gs
            h = o * jnp.tanh(c)
            h_ref[pl.ds(r0 + c0, cr), :] = h
            c_ref[pl.ds(r0 + c0, cr), :] = c


def kernel(features, descendants, parents, W_iou, b_iou, U_iou, W_f, b_f,
           U_f):
    del descendants, parents
    B, Nn, D = features.shape
    H = U_f.shape[0]
    featT = jnp.transpose(features, (1, 0, 2)).reshape(Nn * B, D)
    sc = jnp.concatenate([jnp.full((2 * H,), 0.5, jnp.float32),
                          jnp.ones((H,), jnp.float32)])
    h_t, c_t = pl.pallas_call(
        _tree_kernel,
        out_shape=[jax.ShapeDtypeStruct((Nn * B, H), jnp.float32)] * 2,
    )(featT, (W_iou.T * sc).astype(jnp.bfloat16),
      (b_iou * sc).reshape(1, -1),
      (U_iou.T * sc).astype(jnp.bfloat16), W_f.T.astype(jnp.bfloat16),
      b_f.reshape(1, -1), U_f.T.astype(jnp.bfloat16))
    h = h_t.reshape(Nn, B, H).transpose(1, 0, 2)
    c = c_t.reshape(Nn, B, H).transpose(1, 0, 2)
    return (h, c)
